# Initial kernel scaffold; baseline (speedup 1.0000x reference)
import functools

import jax
import jax.numpy as jnp
from jax import lax
from jax.experimental import pallas as pl
from jax.experimental.pallas import tpu as pltpu
from jax.experimental.pallas import tpu_sc as plsc

NC = 2
NS = 16
NW = NC * NS
CHUNK = 128

_sc_mesh = plsc.VectorSubcoreMesh(
    core_axis_name="c", subcore_axis_name="s", num_cores=NC, num_subcores=NS)


def _worker_id():
  return lax.axis_index("s") * NC + lax.axis_index("c")


def _deg_body(n_pad, ch, dst_hbm, deg0_hbm, deg1_hbm,
              dst_v, ones_v, zero_v, deg_sp, sem):
  del sem
  cid = lax.axis_index("c")
  sid = lax.axis_index("s")
  wid = _worker_id()
  nz = n_pad // NS
  for i in range(CHUNK // 16):
    ones_v[pl.ds(i * 16, 16)] = jnp.ones((16,), jnp.float32)
  for i in range(nz // 16):
    zero_v[pl.ds(i * 16, 16)] = jnp.zeros((16,), jnp.float32)
  pltpu.sync_copy(zero_v, deg_sp.at[pl.ds(sid * nz, nz)])
  plsc.subcore_barrier()
  pltpu.sync_copy(dst_hbm.at[pl.ds(wid * ch, ch)], dst_v)

  def step(i, carry):
    pltpu.sync_copy(ones_v, deg_sp.at[dst_v.at[i]], add=True)
    return carry

  lax.fori_loop(0, ch, step, 0)
  plsc.subcore_barrier()

  @pl.when(cid == 0)
  def _():
    pltpu.sync_copy(deg_sp.at[pl.ds(sid * nz, nz)],
                    deg0_hbm.at[pl.ds(sid * nz, nz)])

  @pl.when(cid == 1)
  def _():
    pltpu.sync_copy(deg_sp.at[pl.ds(sid * nz, nz)],
                    deg1_hbm.at[pl.ds(sid * nz, nz)])


def _edge_body(n_pad, ch, d, g_hbm, src_hbm, dst_hbm, s0_hbm, s1_hbm,
               src_v, dst_v, rows_v, zrows_v, acc_sp, gsem):
  cid = lax.axis_index("c")
  sid = lax.axis_index("s")
  wid = _worker_id()
  rows_per_tile = n_pad // NS
  zr = zrows_v.shape[0]
  for r in range(zr):
    for i in range(d // 16):
      zrows_v[r, pl.ds(i * 16, 16)] = jnp.zeros((16,), jnp.float32)
  for k in range(rows_per_tile // zr):
    pltpu.sync_copy(zrows_v, acc_sp.at[pl.ds(sid * rows_per_tile + k * zr, zr)])
  plsc.subcore_barrier()
  pltpu.sync_copy(src_hbm.at[pl.ds(wid * ch, ch)], src_v)
  pltpu.sync_copy(dst_hbm.at[pl.ds(wid * ch, ch)], dst_v)
  pltpu.async_copy(g_hbm.at[src_v.at[0]], rows_v.at[0], gsem)

  def step(i, carry):
    cur = lax.rem(i, 2)
    nxt = 1 - cur
    pltpu.make_async_copy(g_hbm.at[src_v.at[i]], rows_v.at[cur], gsem).wait()

    @pl.when(i + 1 < ch)
    def _():
      pltpu.async_copy(g_hbm.at[src_v.at[i + 1]], rows_v.at[nxt], gsem)

    pltpu.sync_copy(rows_v.at[cur], acc_sp.at[dst_v.at[i]], add=True)
    return carry

  lax.fori_loop(0, ch, step, 0)
  plsc.subcore_barrier()
  row0 = sid * rows_per_tile

  @pl.when(cid == 0)
  def _():
    pltpu.sync_copy(acc_sp.at[pl.ds(row0, rows_per_tile)],
                    s0_hbm.at[pl.ds(row0, rows_per_tile)])

  @pl.when(cid == 1)
  def _():
    pltpu.sync_copy(acc_sp.at[pl.ds(row0, rows_per_tile)],
                    s1_hbm.at[pl.ds(row0, rows_per_tile)])


def _scale_matmul_body(x_ref, w_ref, d0_ref, d1_ref, g_ref, dinv_ref):
  dinv = lax.rsqrt(d0_ref[...] + d1_ref[...] + 1.0)
  h = jnp.dot(x_ref[...], w_ref[...], preferred_element_type=jnp.float32)
  g_ref[...] = h * dinv
  dinv_ref[...] = dinv


def _combine_body(s0_ref, s1_ref, g_ref, dinv_ref, b_ref, out_ref):
  out_ref[...] = dinv_ref[...] * (s0_ref[...] + s1_ref[...] + g_ref[...]) \
      + b_ref[...]


def kernel(x, edge_index, W, b):
  n, d_in = x.shape
  d_out = W.shape[1]
  e = edge_index.shape[1]
  n_pad = ((n + NW * 10 - 1) // (NW * 10)) * (NW * 10)
  per_w = ((e + NW * CHUNK - 1) // (NW * CHUNK)) * CHUNK
  ch = per_w // CHUNK
  e_pad = per_w * NW

  src = edge_index[0]
  dst = edge_index[1]
  pad = e_pad - e
  if pad:
    pad_ids = jnp.arange(pad, dtype=jnp.int32)
    src = jnp.concatenate([src, pad_ids % min(CHUNK, n)])
    dst = jnp.concatenate([dst, n + pad_ids % (n_pad - n)])
  src2d = src.reshape(NW * ch, CHUNK)
  dst2d = dst.reshape(NW * ch, CHUNK)

  deg_fn = pl.kernel(
      functools.partial(_deg_body, n_pad, ch),
      out_type=(jax.ShapeDtypeStruct((n_pad,), jnp.float32),
                jax.ShapeDtypeStruct((n_pad,), jnp.float32)),
      mesh=_sc_mesh,
      scratch_types=[
          pltpu.VMEM((ch, CHUNK), jnp.int32),
          pltpu.VMEM((CHUNK,), jnp.float32),
          pltpu.VMEM((n_pad // NS,), jnp.float32),
          pltpu.VMEM_SHARED((n_pad,), jnp.float32),
          pltpu.SemaphoreType.DMA,
      ],
  )
  deg0, deg1 = deg_fn(dst2d)

  bm = 256
  grid = n_pad // bm
  g, dinv = pl.pallas_call(
      _scale_matmul_body,
      grid=(grid,),
      in_specs=[
          pl.BlockSpec((bm, d_in), lambda i: (i, 0)),
          pl.BlockSpec((d_in, d_out), lambda i: (0, 0)),
          pl.BlockSpec((bm, 1), lambda i: (i, 0)),
          pl.BlockSpec((bm, 1), lambda i: (i, 0)),
      ],
      out_specs=[
          pl.BlockSpec((bm, d_out), lambda i: (i, 0)),
          pl.BlockSpec((bm, 1), lambda i: (i, 0)),
      ],
      out_shape=[
          jax.ShapeDtypeStruct((n, d_out), jnp.float32),
          jax.ShapeDtypeStruct((n_pad, 1), jnp.float32),
      ],
  )(x, W, deg0.reshape(n_pad, 1), deg1.reshape(n_pad, 1))

  edge_fn = pl.kernel(
      functools.partial(_edge_body, n_pad, ch, d_out),
      out_type=(jax.ShapeDtypeStruct((n_pad, d_out), jnp.float32),
                jax.ShapeDtypeStruct((n_pad, d_out), jnp.float32)),
      mesh=_sc_mesh,
      scratch_types=[
          pltpu.VMEM((ch, CHUNK), jnp.int32),
          pltpu.VMEM((ch, CHUNK), jnp.int32),
          pltpu.VMEM((2, CHUNK, d_out), jnp.float32),
          pltpu.VMEM((40, d_out), jnp.float32),
          pltpu.VMEM_SHARED((n_pad, d_out), jnp.float32),
          pltpu.SemaphoreType.DMA,
      ],
  )
  s0, s1 = edge_fn(g, src2d, dst2d)

  out = pl.pallas_call(
      _combine_body,
      grid=(grid,),
      in_specs=[
          pl.BlockSpec((bm, d_out), lambda i: (i, 0)),
          pl.BlockSpec((bm, d_out), lambda i: (i, 0)),
          pl.BlockSpec((bm, d_out), lambda i: (i, 0)),
          pl.BlockSpec((bm, 1), lambda i: (i, 0)),
          pl.BlockSpec((1, d_out), lambda i: (0, 0)),
      ],
      out_specs=pl.BlockSpec((bm, d_out), lambda i: (i, 0)),
      out_shape=jax.ShapeDtypeStruct((n, d_out), jnp.float32),
  )(s0, s1, g, dinv, b.reshape(1, d_out))
  return out


# speedup vs baseline: 32.5304x; 32.5304x over previous
import functools

import jax
import jax.numpy as jnp
from jax import lax
from jax.experimental import pallas as pl
from jax.experimental.pallas import tpu as pltpu
from jax.experimental.pallas import tpu_sc as plsc

NC = 2
NS = 16
NW = NC * NS
CHUNK = 128

_sc_mesh = plsc.VectorSubcoreMesh(
    core_axis_name="c", subcore_axis_name="s", num_cores=NC, num_subcores=NS)


def _worker_id():
  return lax.axis_index("s") * NC + lax.axis_index("c")


def _deg_body(n_pad, ch, dst_hbm, deg0_hbm, deg1_hbm,
              dst_v, ones_v, zero_v, deg_sp, sem):
  del sem
  cid = lax.axis_index("c")
  sid = lax.axis_index("s")
  wid = _worker_id()
  nz = n_pad // NS
  for i in range(CHUNK // 16):
    ones_v[pl.ds(i * 16, 16)] = jnp.ones((16,), jnp.float32)
  for i in range(nz // 16):
    zero_v[pl.ds(i * 16, 16)] = jnp.zeros((16,), jnp.float32)
  pltpu.sync_copy(zero_v, deg_sp.at[pl.ds(sid * nz, nz)])
  plsc.subcore_barrier()
  pltpu.sync_copy(dst_hbm.at[pl.ds(wid * ch, ch)], dst_v)

  def step(i, carry):
    pltpu.sync_copy(ones_v, deg_sp.at[dst_v.at[i]], add=True)
    return carry

  lax.fori_loop(0, ch, step, 0)
  plsc.subcore_barrier()

  @pl.when(cid == 0)
  def _():
    pltpu.sync_copy(deg_sp.at[pl.ds(sid * nz, nz)],
                    deg0_hbm.at[pl.ds(sid * nz, nz)])

  @pl.when(cid == 1)
  def _():
    pltpu.sync_copy(deg_sp.at[pl.ds(sid * nz, nz)],
                    deg1_hbm.at[pl.ds(sid * nz, nz)])


def _edge_body(n_pad, ch, d, g_hbm, src_hbm, dst_hbm, s0_hbm, s1_hbm,
               src_v, dst_v, rows_v, zrows_v, acc_sp, gsem):
  cid = lax.axis_index("c")
  sid = lax.axis_index("s")
  wid = _worker_id()
  rows_per_tile = n_pad // NS
  iw = src_v.shape[0]
  zr = zrows_v.shape[0]
  for r in range(zr):
    for i in range(d // 16):
      zrows_v[r, pl.ds(i * 16, 16)] = jnp.zeros((16,), jnp.float32)
  for k in range(rows_per_tile // zr):
    pltpu.sync_copy(zrows_v, acc_sp.at[pl.ds(sid * rows_per_tile + k * zr, zr)])
  plsc.subcore_barrier()

  def window(w, carry):
    base = wid * ch + w * iw
    pltpu.sync_copy(src_hbm.at[pl.ds(base, iw)], src_v)
    pltpu.sync_copy(dst_hbm.at[pl.ds(base, iw)], dst_v)
    pltpu.async_copy(g_hbm.at[src_v.at[0]], rows_v.at[0], gsem)

    def step(i, carry2):
      cur = lax.rem(i, 2)
      nxt = 1 - cur
      pltpu.make_async_copy(g_hbm.at[src_v.at[i]], rows_v.at[cur], gsem).wait()

      @pl.when(i + 1 < iw)
      def _():
        pltpu.async_copy(g_hbm.at[src_v.at[i + 1]], rows_v.at[nxt], gsem)

      pltpu.sync_copy(rows_v.at[cur], acc_sp.at[dst_v.at[i]], add=True)
      return carry2

    return lax.fori_loop(0, iw, step, carry)

  lax.fori_loop(0, ch // iw, window, 0)
  plsc.subcore_barrier()
  row0 = sid * rows_per_tile

  @pl.when(cid == 0)
  def _():
    pltpu.sync_copy(acc_sp.at[pl.ds(row0, rows_per_tile)],
                    s0_hbm.at[pl.ds(row0, rows_per_tile)])

  @pl.when(cid == 1)
  def _():
    pltpu.sync_copy(acc_sp.at[pl.ds(row0, rows_per_tile)],
                    s1_hbm.at[pl.ds(row0, rows_per_tile)])


def _scale_matmul_body(x_ref, w_ref, d0_ref, d1_ref, g_ref, dinv_ref):
  dinv = lax.rsqrt(d0_ref[...] + d1_ref[...] + 1.0)
  h = jnp.dot(x_ref[...], w_ref[...], preferred_element_type=jnp.float32)
  g_ref[...] = h * dinv
  dinv_ref[...] = dinv


def _combine_body(s0_ref, s1_ref, g_ref, dinv_ref, b_ref, out_ref):
  out_ref[...] = dinv_ref[...] * (s0_ref[...] + s1_ref[...] + g_ref[...]) \
      + b_ref[...]


def kernel(x, edge_index, W, b):
  n, d_in = x.shape
  d_out = W.shape[1]
  e = edge_index.shape[1]
  n_pad = ((n + NW * 10 - 1) // (NW * 10)) * (NW * 10)
  per_w = ((e + NW * CHUNK * 8 - 1) // (NW * CHUNK * 8)) * CHUNK * 8
  ch = per_w // CHUNK
  e_pad = per_w * NW

  src = edge_index[0]
  dst = edge_index[1]
  pad = e_pad - e
  if pad:
    pad_ids = jnp.arange(pad, dtype=jnp.int32)
    src = jnp.concatenate([src, pad_ids % min(CHUNK, n)])
    dst = jnp.concatenate([dst, n + pad_ids % (n_pad - n)])
  src2d = src.reshape(NW * ch, CHUNK)
  dst2d = dst.reshape(NW * ch, CHUNK)

  deg_fn = pl.kernel(
      functools.partial(_deg_body, n_pad, ch),
      out_type=(jax.ShapeDtypeStruct((n_pad,), jnp.float32),
                jax.ShapeDtypeStruct((n_pad,), jnp.float32)),
      mesh=_sc_mesh,
      scratch_types=[
          pltpu.VMEM((ch, CHUNK), jnp.int32),
          pltpu.VMEM((CHUNK,), jnp.float32),
          pltpu.VMEM((n_pad // NS,), jnp.float32),
          pltpu.VMEM_SHARED((n_pad,), jnp.float32),
          pltpu.SemaphoreType.DMA,
      ],
  )
  deg0, deg1 = deg_fn(dst2d)

  bm = 256
  grid = n_pad // bm
  g, dinv = pl.pallas_call(
      _scale_matmul_body,
      grid=(grid,),
      in_specs=[
          pl.BlockSpec((bm, d_in), lambda i: (i, 0)),
          pl.BlockSpec((d_in, d_out), lambda i: (0, 0)),
          pl.BlockSpec((bm, 1), lambda i: (i, 0)),
          pl.BlockSpec((bm, 1), lambda i: (i, 0)),
      ],
      out_specs=[
          pl.BlockSpec((bm, d_out), lambda i: (i, 0)),
          pl.BlockSpec((bm, 1), lambda i: (i, 0)),
      ],
      out_shape=[
          jax.ShapeDtypeStruct((n, d_out), jnp.float32),
          jax.ShapeDtypeStruct((n_pad, 1), jnp.float32),
      ],
  )(x, W, deg0.reshape(n_pad, 1), deg1.reshape(n_pad, 1))

  edge_fn = pl.kernel(
      functools.partial(_edge_body, n_pad, ch, d_out),
      out_type=(jax.ShapeDtypeStruct((n_pad, d_out), jnp.float32),
                jax.ShapeDtypeStruct((n_pad, d_out), jnp.float32)),
      mesh=_sc_mesh,
      scratch_types=[
          pltpu.VMEM((16, CHUNK), jnp.int32),
          pltpu.VMEM((16, CHUNK), jnp.int32),
          pltpu.VMEM((2, CHUNK, d_out), jnp.float32),
          pltpu.VMEM((8, d_out), jnp.float32),
          pltpu.VMEM_SHARED((n_pad, d_out), jnp.float32),
          pltpu.SemaphoreType.DMA,
      ],
  )
  s0, s1 = edge_fn(g, src2d, dst2d)

  out = pl.pallas_call(
      _combine_body,
      grid=(grid,),
      in_specs=[
          pl.BlockSpec((bm, d_out), lambda i: (i, 0)),
          pl.BlockSpec((bm, d_out), lambda i: (i, 0)),
          pl.BlockSpec((bm, d_out), lambda i: (i, 0)),
          pl.BlockSpec((bm, 1), lambda i: (i, 0)),
          pl.BlockSpec((1, d_out), lambda i: (0, 0)),
      ],
      out_specs=pl.BlockSpec((bm, d_out), lambda i: (i, 0)),
      out_shape=jax.ShapeDtypeStruct((n, d_out), jnp.float32),
  )(s0, s1, g, dinv, b.reshape(1, d_out))
  return out


# speedup vs baseline: 36.2520x; 1.1144x over previous
import functools

import jax
import jax.numpy as jnp
import numpy as np
from jax import lax
from jax.experimental import pallas as pl
from jax.experimental.pallas import tpu as pltpu
from jax.experimental.pallas import tpu_sc as plsc

NC = 2
NS = 16
NW = NC * NS
CHUNK = 128

_sc_mesh = plsc.VectorSubcoreMesh(
    core_axis_name="c", subcore_axis_name="s", num_cores=NC, num_subcores=NS)


def _worker_id():
  return lax.axis_index("s") * NC + lax.axis_index("c")


def _deg_body(n_pad, ch, dst_hbm, deg0_hbm, deg1_hbm,
              dst_v, ones_v, zero_v, deg_sp, sem):
  cid = lax.axis_index("c")
  sid = lax.axis_index("s")
  wid = _worker_id()
  nz = n_pad // NS
  for i in range(CHUNK // 16):
    ones_v[pl.ds(i * 16, 16)] = jnp.ones((16,), jnp.float32)
  for i in range(nz // 16):
    zero_v[pl.ds(i * 16, 16)] = jnp.zeros((16,), jnp.float32)
  pltpu.sync_copy(zero_v, deg_sp.at[pl.ds(sid * nz, nz)])
  plsc.subcore_barrier()
  pltpu.sync_copy(dst_hbm.at[pl.ds(wid * ch, ch)], dst_v)

  def step(i, carry):
    pltpu.async_copy(ones_v, deg_sp.at[dst_v.at[i]], sem, add=True)
    return carry

  lax.fori_loop(0, ch, step, 0)

  def drain(i, carry):
    pltpu.make_async_copy(ones_v, deg_sp.at[dst_v.at[i]], sem).wait()
    return carry

  lax.fori_loop(0, ch, drain, 0)
  plsc.subcore_barrier()

  @pl.when(cid == 0)
  def _():
    pltpu.sync_copy(deg_sp.at[pl.ds(sid * nz, nz)],
                    deg0_hbm.at[pl.ds(sid * nz, nz)])

  @pl.when(cid == 1)
  def _():
    pltpu.sync_copy(deg_sp.at[pl.ds(sid * nz, nz)],
                    deg1_hbm.at[pl.ds(sid * nz, nz)])


def _edge_body(n_pad, ch, d, g_hbm, src_hbm, dst_hbm, s0_hbm, s1_hbm,
               src_v, dst_v, rows_v, zrows_v, acc_sp, gsem):
  cid = lax.axis_index("c")
  sid = lax.axis_index("s")
  wid = _worker_id()
  rows_per_tile = n_pad // NS
  iw = src_v.shape[0]
  zr = zrows_v.shape[0]
  for r in range(zr):
    for i in range(d // 16):
      zrows_v[r, pl.ds(i * 16, 16)] = jnp.zeros((16,), jnp.float32)
  for k in range(rows_per_tile // zr):
    pltpu.sync_copy(zrows_v, acc_sp.at[pl.ds(sid * rows_per_tile + k * zr, zr)])
  plsc.subcore_barrier()

  def window(w, carry):
    base = wid * ch + w * iw
    pltpu.sync_copy(src_hbm.at[pl.ds(base, iw)], src_v)
    pltpu.sync_copy(dst_hbm.at[pl.ds(base, iw)], dst_v)
    pltpu.async_copy(g_hbm.at[src_v.at[0]], rows_v.at[0], gsem)

    def step(i, carry2):
      cur = lax.rem(i, 2)
      nxt = 1 - cur
      pltpu.make_async_copy(g_hbm.at[src_v.at[i]], rows_v.at[cur], gsem).wait()

      @pl.when(i + 1 < iw)
      def _():
        pltpu.async_copy(g_hbm.at[src_v.at[i + 1]], rows_v.at[nxt], gsem)

      pltpu.sync_copy(rows_v.at[cur], acc_sp.at[dst_v.at[i]], add=True)
      return carry2

    return lax.fori_loop(0, iw, step, carry)

  lax.fori_loop(0, ch // iw, window, 0)
  plsc.subcore_barrier()
  row0 = sid * rows_per_tile

  @pl.when(cid == 0)
  def _():
    pltpu.sync_copy(acc_sp.at[pl.ds(row0, rows_per_tile)],
                    s0_hbm.at[pl.ds(row0, rows_per_tile)])

  @pl.when(cid == 1)
  def _():
    pltpu.sync_copy(acc_sp.at[pl.ds(row0, rows_per_tile)],
                    s1_hbm.at[pl.ds(row0, rows_per_tile)])


def _scale_matmul_body(x_ref, w_ref, d0_ref, d1_ref, g_ref, dinv_ref):
  dinv = lax.rsqrt(d0_ref[...] + d1_ref[...] + 1.0)
  h = jnp.dot(x_ref[...], w_ref[...], preferred_element_type=jnp.float32)
  g_ref[...] = h * dinv
  dinv_ref[...] = dinv


def _combine_body(s0_ref, s1_ref, g_ref, dinv_ref, b_ref, out_ref):
  out_ref[...] = dinv_ref[...] * (s0_ref[...] + s1_ref[...] + g_ref[...]) \
      + b_ref[...]


def kernel(x, edge_index, W, b):
  n, d_in = x.shape
  d_out = W.shape[1]
  e = edge_index.shape[1]
  n_pad = ((n + NW * 10 - 1) // (NW * 10)) * (NW * 10)
  per_w = ((e + NW * CHUNK * 8 - 1) // (NW * CHUNK * 8)) * CHUNK * 8
  ch = per_w // CHUNK
  e_pad = per_w * NW

  src = edge_index[0]
  dst = edge_index[1]
  pad = e_pad - e
  if pad:
    pad_ids = np.arange(pad, dtype=np.int32)
    src = jnp.concatenate([src, jnp.asarray(pad_ids % min(CHUNK, n))])
    dst = jnp.concatenate([dst, jnp.asarray(n + pad_ids % (n_pad - n))])
  src2d = src.reshape(NW * ch, CHUNK)
  dst2d = dst.reshape(NW * ch, CHUNK)

  deg_fn = pl.kernel(
      functools.partial(_deg_body, n_pad, ch),
      out_type=(jax.ShapeDtypeStruct((n_pad,), jnp.float32),
                jax.ShapeDtypeStruct((n_pad,), jnp.float32)),
      mesh=_sc_mesh,
      scratch_types=[
          pltpu.VMEM((ch, CHUNK), jnp.int32),
          pltpu.VMEM((CHUNK,), jnp.float32),
          pltpu.VMEM((n_pad // NS,), jnp.float32),
          pltpu.VMEM_SHARED((n_pad,), jnp.float32),
          pltpu.SemaphoreType.DMA,
      ],
  )
  deg0, deg1 = deg_fn(dst2d)

  bm = 512
  grid = n_pad // bm
  g, dinv = pl.pallas_call(
      _scale_matmul_body,
      grid=(grid,),
      in_specs=[
          pl.BlockSpec((bm, d_in), lambda i: (i, 0)),
          pl.BlockSpec((d_in, d_out), lambda i: (0, 0)),
          pl.BlockSpec((bm, 1), lambda i: (i, 0)),
          pl.BlockSpec((bm, 1), lambda i: (i, 0)),
      ],
      out_specs=[
          pl.BlockSpec((bm, d_out), lambda i: (i, 0)),
          pl.BlockSpec((bm, 1), lambda i: (i, 0)),
      ],
      out_shape=[
          jax.ShapeDtypeStruct((n, d_out), jnp.float32),
          jax.ShapeDtypeStruct((n_pad, 1), jnp.float32),
      ],
  )(x, W, deg0.reshape(n_pad, 1), deg1.reshape(n_pad, 1))

  edge_fn = pl.kernel(
      functools.partial(_edge_body, n_pad, ch, d_out),
      out_type=(jax.ShapeDtypeStruct((n_pad, d_out), jnp.float32),
                jax.ShapeDtypeStruct((n_pad, d_out), jnp.float32)),
      mesh=_sc_mesh,
      scratch_types=[
          pltpu.VMEM((16, CHUNK), jnp.int32),
          pltpu.VMEM((16, CHUNK), jnp.int32),
          pltpu.VMEM((2, CHUNK, d_out), jnp.float32),
          pltpu.VMEM((8, d_out), jnp.float32),
          pltpu.VMEM_SHARED((n_pad, d_out), jnp.float32),
          pltpu.SemaphoreType.DMA,
      ],
  )
  s0, s1 = edge_fn(g, src2d, dst2d)

  out = pl.pallas_call(
      _combine_body,
      grid=(grid,),
      in_specs=[
          pl.BlockSpec((bm, d_out), lambda i: (i, 0)),
          pl.BlockSpec((bm, d_out), lambda i: (i, 0)),
          pl.BlockSpec((bm, d_out), lambda i: (i, 0)),
          pl.BlockSpec((bm, 1), lambda i: (i, 0)),
          pl.BlockSpec((1, d_out), lambda i: (0, 0)),
      ],
      out_specs=pl.BlockSpec((bm, d_out), lambda i: (i, 0)),
      out_shape=jax.ShapeDtypeStruct((n, d_out), jnp.float32),
  )(s0, s1, g, dinv, b.reshape(1, d_out))
  return out
